# baseline (speedup 1.0000x reference)
import functools

import jax
import jax.numpy as jnp
from jax import lax
from jax.experimental import pallas as pl
from jax.experimental.pallas import tpu as pltpu
from jax.experimental.pallas import tpu_sc as plsc

BATCH = 16384
EMBED = 32
VOCAB = 1_000_000

_info = plsc.get_sparse_core_info()
NC = _info.num_cores
NS = _info.num_subcores
LANES = _info.num_lanes
NW = NC * NS
BPW = BATCH // NW
CHUNK = 128
NCHUNK = BPW // CHUNK

CONV_C = 2048


NBLK = -(-VOCAB // CONV_C)
VOCAB_PAD = NBLK * CONV_C


def _detile_body(tabT_ref, out_ref):
    eye = jnp.eye(EMBED, dtype=jnp.float32)
    for k in range(4):
        x_k = tabT_ref[:, pl.ds(k * (CONV_C // 4), CONV_C // 4)]
        out_ref[:, pl.ds(k * EMBED, EMBED)] = jax.lax.dot_general(
            x_k, eye, (((0,), (0,)), ((), ())),
            preferred_element_type=jnp.float32)


def _to_linear(tabT):
    out2d = pl.pallas_call(
        _detile_body,
        grid=(NBLK,),
        in_specs=[pl.BlockSpec((EMBED, CONV_C), lambda i: (0, i))],
        out_specs=pl.BlockSpec((CONV_C // 4, 128), lambda i: (i, 0)),
        out_shape=jax.ShapeDtypeStruct((VOCAB_PAD * EMBED // 128, 128), jnp.float32),
    )(tabT)
    return out2d.reshape(VOCAB_PAD * EMBED)


def _gmf_body(users_hbm, items_hbm, utab_hbm, itab_hbm, out_hbm,
              idx_u, idx_i, urows, irows, orows, sem):
    wid = lax.axis_index("s") * NC + lax.axis_index("c")
    row0 = wid * NCHUNK

    pltpu.sync_copy(users_hbm.at[pl.ds(row0, NCHUNK)], idx_u)
    pltpu.sync_copy(items_hbm.at[pl.ds(row0, NCHUNK)], idx_i)

    for idx_ref in (idx_u, idx_i):
        for j in range(NCHUNK):
            for s in range(CHUNK // LANES):
                v = idx_ref[j, pl.ds(s * LANES, LANES)]
                idx_ref[j, pl.ds(s * LANES, LANES)] = (
                    ((v >> 11) << 11) + ((v & 511) << 2) + ((v >> 9) & 3))

    copies = []
    for j in range(NCHUNK):
        copies.append(pltpu.async_copy(
            utab_hbm.at[idx_u.at[j]], urows.at[pl.ds(j * CHUNK, CHUNK)], sem))
        copies.append(pltpu.async_copy(
            itab_hbm.at[idx_i.at[j]], irows.at[pl.ds(j * CHUNK, CHUNK)], sem))
    for c in copies:
        c.wait()

    def mul_row(b, carry):
        orows[b, pl.ds(0, LANES)] = urows[b, pl.ds(0, LANES)] * irows[b, pl.ds(0, LANES)]
        orows[b, pl.ds(LANES, LANES)] = urows[b, pl.ds(LANES, LANES)] * irows[b, pl.ds(LANES, LANES)]
        return carry
    lax.fori_loop(0, BPW, mul_row, 0)

    pltpu.sync_copy(orows, out_hbm.at[pl.ds(wid * BPW, BPW)])


def kernel(users, items, user_table, item_table):
    users2 = users.astype(jnp.int32).reshape(NW * NCHUNK, CHUNK)
    items2 = items.astype(jnp.int32).reshape(NW * NCHUNK, CHUNK)
    utab = _to_linear(user_table.T).reshape(VOCAB_PAD, EMBED)
    itab = _to_linear(item_table.T).reshape(VOCAB_PAD, EMBED)
    run = pl.kernel(
        _gmf_body,
        out_type=jax.ShapeDtypeStruct((BATCH, EMBED), jnp.float32),
        mesh=plsc.VectorSubcoreMesh(core_axis_name="c", subcore_axis_name="s"),
        scratch_types=[
            pltpu.VMEM((NCHUNK, CHUNK), jnp.int32),
            pltpu.VMEM((NCHUNK, CHUNK), jnp.int32),
            pltpu.VMEM((BPW, EMBED), jnp.float32),
            pltpu.VMEM((BPW, EMBED), jnp.float32),
            pltpu.VMEM((BPW, EMBED), jnp.float32),
            pltpu.SemaphoreType.DMA,
        ],
        compiler_params=pltpu.CompilerParams(use_tc_tiling_on_sc=False),
    )
    return run(users2, items2, utab, itab)


# speedup vs baseline: 1.6267x; 1.6267x over previous
import functools

import jax
import jax.numpy as jnp
from jax import lax
from jax.experimental import pallas as pl
from jax.experimental.pallas import tpu as pltpu
from jax.experimental.pallas import tpu_sc as plsc

BATCH = 16384
EMBED = 32
VOCAB = 1_000_000

_info = plsc.get_sparse_core_info()
NC = _info.num_cores
NS = _info.num_subcores
LANES = _info.num_lanes
NW = NC * NS
BPW = BATCH // NW
CHUNK = 128
NCHUNK = BPW // CHUNK

CONV_C = 8192
QUART = CONV_C // 4
C_SH = CONV_C.bit_length() - 1
Q_SH = QUART.bit_length() - 1
NBLK = -(-VOCAB // CONV_C)
VOCAB_PAD = NBLK * CONV_C


def _detile_body(tabT_ref, out_ref):
    for k in range(4):
        out_ref[:, pl.ds(k * EMBED, EMBED)] = jnp.transpose(
            tabT_ref[:, pl.ds(k * (CONV_C // 4), CONV_C // 4)], (1, 0))


def _to_linear(tabT):
    out2d = pl.pallas_call(
        _detile_body,
        grid=(NBLK,),
        in_specs=[pl.BlockSpec((EMBED, CONV_C), lambda i: (0, i))],
        out_specs=pl.BlockSpec((CONV_C // 4, 128), lambda i: (i, 0)),
        out_shape=jax.ShapeDtypeStruct((VOCAB_PAD * EMBED // 128, 128), jnp.float32),
    )(tabT)
    return out2d.reshape(VOCAB_PAD * EMBED)


def _gmf_body(users_hbm, items_hbm, utab_hbm, itab_hbm, out_hbm,
              idx_u, idx_i, urows, irows, orows, sem):
    wid = lax.axis_index("s") * NC + lax.axis_index("c")
    row0 = wid * NCHUNK

    pltpu.sync_copy(users_hbm.at[pl.ds(row0, NCHUNK)], idx_u)
    pltpu.sync_copy(items_hbm.at[pl.ds(row0, NCHUNK)], idx_i)

    for idx_ref in (idx_u, idx_i):
        for j in range(NCHUNK):
            for s in range(CHUNK // LANES):
                v = idx_ref[j, pl.ds(s * LANES, LANES)]
                idx_ref[j, pl.ds(s * LANES, LANES)] = (
                    ((v >> C_SH) << C_SH)
                    + ((v & (QUART - 1)) << 2)
                    + ((v >> Q_SH) & 3))

    copies = []
    for j in range(NCHUNK):
        copies.append(pltpu.async_copy(
            utab_hbm.at[idx_u.at[j]], urows.at[pl.ds(j * CHUNK, CHUNK)], sem))
        copies.append(pltpu.async_copy(
            itab_hbm.at[idx_i.at[j]], irows.at[pl.ds(j * CHUNK, CHUNK)], sem))
    for c in copies:
        c.wait()

    def mul_row(b, carry):
        orows[b, pl.ds(0, LANES)] = urows[b, pl.ds(0, LANES)] * irows[b, pl.ds(0, LANES)]
        orows[b, pl.ds(LANES, LANES)] = urows[b, pl.ds(LANES, LANES)] * irows[b, pl.ds(LANES, LANES)]
        return carry
    lax.fori_loop(0, BPW, mul_row, 0)

    pltpu.sync_copy(orows, out_hbm.at[pl.ds(wid * BPW, BPW)])


def kernel(users, items, user_table, item_table):
    users2 = users.astype(jnp.int32).reshape(NW * NCHUNK, CHUNK)
    items2 = items.astype(jnp.int32).reshape(NW * NCHUNK, CHUNK)
    utab = _to_linear(user_table.T).reshape(VOCAB_PAD, EMBED)
    itab = _to_linear(item_table.T).reshape(VOCAB_PAD, EMBED)
    run = pl.kernel(
        _gmf_body,
        out_type=jax.ShapeDtypeStruct((BATCH, EMBED), jnp.float32),
        mesh=plsc.VectorSubcoreMesh(core_axis_name="c", subcore_axis_name="s"),
        scratch_types=[
            pltpu.VMEM((NCHUNK, CHUNK), jnp.int32),
            pltpu.VMEM((NCHUNK, CHUNK), jnp.int32),
            pltpu.VMEM((BPW, EMBED), jnp.float32),
            pltpu.VMEM((BPW, EMBED), jnp.float32),
            pltpu.VMEM((BPW, EMBED), jnp.float32),
            pltpu.SemaphoreType.DMA,
        ],
        compiler_params=pltpu.CompilerParams(use_tc_tiling_on_sc=False),
    )
    return run(users2, items2, utab, itab)


# speedup vs baseline: 1.6688x; 1.0259x over previous
import functools

import jax
import jax.numpy as jnp
from jax import lax
from jax.experimental import pallas as pl
from jax.experimental.pallas import tpu as pltpu
from jax.experimental.pallas import tpu_sc as plsc

BATCH = 16384
EMBED = 32
VOCAB = 1_000_000

_info = plsc.get_sparse_core_info()
NC = _info.num_cores
NS = _info.num_subcores
LANES = _info.num_lanes
NW = NC * NS
BPW = BATCH // NW
CHUNK = 128
NCHUNK = BPW // CHUNK

CONV_C = 32768
QUART = CONV_C // 4
C_SH = CONV_C.bit_length() - 1
Q_SH = QUART.bit_length() - 1
NBLK = -(-VOCAB // CONV_C)
VOCAB_PAD = NBLK * CONV_C


def _detile_body(tabT_ref, out_ref):
    for k in range(4):
        out_ref[:, pl.ds(k * EMBED, EMBED)] = jnp.transpose(
            tabT_ref[:, pl.ds(k * (CONV_C // 4), CONV_C // 4)], (1, 0))


def _to_linear(tabT):
    out2d = pl.pallas_call(
        _detile_body,
        grid=(NBLK,),
        in_specs=[pl.BlockSpec((EMBED, CONV_C), lambda i: (0, i))],
        out_specs=pl.BlockSpec((CONV_C // 4, 128), lambda i: (i, 0)),
        out_shape=jax.ShapeDtypeStruct((VOCAB_PAD * EMBED // 128, 128), jnp.float32),
    )(tabT)
    return out2d.reshape(VOCAB_PAD * EMBED)


def _gmf_body(users_hbm, items_hbm, utab_hbm, itab_hbm, out_hbm,
              idx_u, idx_i, urows, irows, orows, sem):
    wid = lax.axis_index("s") * NC + lax.axis_index("c")
    row0 = wid * NCHUNK

    pltpu.sync_copy(users_hbm.at[pl.ds(row0, NCHUNK)], idx_u)
    pltpu.sync_copy(items_hbm.at[pl.ds(row0, NCHUNK)], idx_i)

    for idx_ref in (idx_u, idx_i):
        for j in range(NCHUNK):
            for s in range(CHUNK // LANES):
                v = idx_ref[j, pl.ds(s * LANES, LANES)]
                idx_ref[j, pl.ds(s * LANES, LANES)] = (
                    ((v >> C_SH) << C_SH)
                    + ((v & (QUART - 1)) << 2)
                    + ((v >> Q_SH) & 3))

    copies = []
    for j in range(NCHUNK):
        copies.append(pltpu.async_copy(
            utab_hbm.at[idx_u.at[j]], urows.at[pl.ds(j * CHUNK, CHUNK)], sem))
        copies.append(pltpu.async_copy(
            itab_hbm.at[idx_i.at[j]], irows.at[pl.ds(j * CHUNK, CHUNK)], sem))
    for c in copies:
        c.wait()

    def mul_row(b, carry):
        orows[b, pl.ds(0, LANES)] = urows[b, pl.ds(0, LANES)] * irows[b, pl.ds(0, LANES)]
        orows[b, pl.ds(LANES, LANES)] = urows[b, pl.ds(LANES, LANES)] * irows[b, pl.ds(LANES, LANES)]
        return carry
    lax.fori_loop(0, BPW, mul_row, 0)

    pltpu.sync_copy(orows, out_hbm.at[pl.ds(wid * BPW, BPW)])


def kernel(users, items, user_table, item_table):
    users2 = users.astype(jnp.int32).reshape(NW * NCHUNK, CHUNK)
    items2 = items.astype(jnp.int32).reshape(NW * NCHUNK, CHUNK)
    utab = _to_linear(user_table.T).reshape(VOCAB_PAD, EMBED)
    itab = _to_linear(item_table.T).reshape(VOCAB_PAD, EMBED)
    run = pl.kernel(
        _gmf_body,
        out_type=jax.ShapeDtypeStruct((BATCH, EMBED), jnp.float32),
        mesh=plsc.VectorSubcoreMesh(core_axis_name="c", subcore_axis_name="s"),
        scratch_types=[
            pltpu.VMEM((NCHUNK, CHUNK), jnp.int32),
            pltpu.VMEM((NCHUNK, CHUNK), jnp.int32),
            pltpu.VMEM((BPW, EMBED), jnp.float32),
            pltpu.VMEM((BPW, EMBED), jnp.float32),
            pltpu.VMEM((BPW, EMBED), jnp.float32),
            pltpu.SemaphoreType.DMA,
        ],
        compiler_params=pltpu.CompilerParams(use_tc_tiling_on_sc=False),
    )
    return run(users2, items2, utab, itab)
